# Initial kernel scaffold; baseline (speedup 1.0000x reference)
import jax
import jax.numpy as jnp
from jax.experimental import pallas as pl


def kernel(x, adj_row, adj_col, adj_data, embed, W0, b0, W1, b1, Wg1, Wg2, nodeid, tmp):
    raise NotImplementedError("write your pallas kernel here")


# speedup vs baseline: 4.2255x; 4.2255x over previous
import functools

import jax
import jax.numpy as jnp
from jax import lax
from jax.experimental import pallas as pl
from jax.experimental.pallas import tpu as pltpu
from jax.experimental.pallas import tpu_sc as plsc

NN = 10000
NE = 320000
NE2 = 2 * NE
HD = 64

NT = 10240
NTILES = 32
W = 128
EP = 327680
E2P = 655360
EPT = EP // NTILES
E2PT = E2P // NTILES

@functools.cache
def _sc_mesh():
    return plsc.VectorSubcoreMesh(core_axis_name="c", subcore_axis_name="s",
                                  num_cores=2, num_subcores=16)


def _wid():
    return lax.axis_index("s") * 2 + lax.axis_index("c")


def _proj_body(e_ref, x_ref, w0a_ref, w0b_ref, wg1_ref, px_ref, p2_ref):
    e = e_ref[...]
    px_ref[:, :HD] = jnp.dot(e, w0a_ref[...], preferred_element_type=jnp.float32)
    px_ref[:, HD:] = jnp.dot(x_ref[...], wg1_ref[...],
                             preferred_element_type=jnp.float32)
    p2 = jnp.dot(e, w0b_ref[...], preferred_element_type=jnp.float32)
    p2_ref[:, :HD] = p2
    p2_ref[:, HD:] = p2


def _projections(embed, x, w0a, w0b, wg1):
    bn = 400
    grid = NN // bn
    return pl.pallas_call(
        _proj_body,
        grid=(grid,),
        in_specs=[
            pl.BlockSpec((bn, 128), lambda i: (i, 0)),
            pl.BlockSpec((bn, 128), lambda i: (i, 0)),
            pl.BlockSpec((128, HD), lambda i: (0, 0)),
            pl.BlockSpec((128, HD), lambda i: (0, 0)),
            pl.BlockSpec((128, HD), lambda i: (0, 0)),
        ],
        out_specs=[
            pl.BlockSpec((bn, 2 * HD), lambda i: (i, 0)),
            pl.BlockSpec((bn, 2 * HD), lambda i: (i, 0)),
        ],
        out_shape=[
            jax.ShapeDtypeStruct((NN, 2 * HD), jnp.float32),
            jax.ShapeDtypeStruct((NN, 2 * HD), jnp.float32),
        ],
    )(embed, x, w0a, w0b, wg1)


def _gather_rows_body(px_hbm, p2_hbm, row_hbm, col_hbm, ga_hbm, g2_hbm,
                      idx_v, rows_v, cols_v):
    base0 = _wid() * EPT

    @pl.loop(0, EPT, step=W)
    def _(off):
        base = base0 + off
        pltpu.sync_copy(row_hbm.at[pl.ds(base, W)], idx_v)
        pltpu.sync_copy(px_hbm.at[idx_v], rows_v)
        pltpu.sync_copy(rows_v, ga_hbm.at[pl.ds(base, W)])
        pltpu.sync_copy(col_hbm.at[pl.ds(base, W)], idx_v)
        pltpu.sync_copy(p2_hbm.at[idx_v], cols_v)
        pltpu.sync_copy(cols_v, g2_hbm.at[pl.ds(base, W)])


def _gather_rows(px, p2, rowp, colp):
    return pl.kernel(
        _gather_rows_body,
        out_type=[
            jax.ShapeDtypeStruct((EP, 2 * HD), jnp.float32),
            jax.ShapeDtypeStruct((EP, 2 * HD), jnp.float32),
        ],
        mesh=_sc_mesh(),
        scratch_types=[
            pltpu.VMEM((W,), jnp.int32),
            pltpu.VMEM((W, 2 * HD), jnp.float32),
            pltpu.VMEM((W, 2 * HD), jnp.float32),
        ],
    )(px, p2, rowp, colp)


def _edge_mlp_body(ga_ref, g2_ref, c0row_ref, w0c_ref, b0_ref, w1_ref, b1_ref,
                   v_ref):
    c0 = jnp.dot(c0row_ref[...], w0c_ref[...],
                 preferred_element_type=jnp.float32) + b0_ref[...]
    h = jnp.maximum(ga_ref[:, :HD] + g2_ref[:, :HD] + c0, 0.0)
    z = jnp.dot(h, w1_ref[...], preferred_element_type=jnp.float32) + b1_ref[...]
    v_ref[...] = jax.nn.sigmoid(z)


def _edge_mlp(ga, g2, c0row, w0c, b0, w1, b1):
    be = 2560
    grid = EP // be
    return pl.pallas_call(
        _edge_mlp_body,
        grid=(grid,),
        in_specs=[
            pl.BlockSpec((be, 2 * HD), lambda i: (i, 0)),
            pl.BlockSpec((be, 2 * HD), lambda i: (i, 0)),
            pl.BlockSpec((1, 128), lambda i: (0, 0)),
            pl.BlockSpec((128, HD), lambda i: (0, 0)),
            pl.BlockSpec((1, HD), lambda i: (0, 0)),
            pl.BlockSpec((HD, 1), lambda i: (0, 0)),
            pl.BlockSpec((1, 1), lambda i: (0, 0)),
        ],
        out_specs=pl.BlockSpec((be, 1), lambda i: (i, 0)),
        out_shape=jax.ShapeDtypeStruct((EP, 1), jnp.float32),
    )(ga, g2, c0row, w0c, b0, w1, b1)


def _runid_body(k_ref, rid_ref, carry):
    step = pl.program_id(0)

    @pl.when(step == 0)
    def _():
        carry[0] = 0
        carry[1] = -1

    k = k_ref[...]
    rowhead = jnp.concatenate(
        [jnp.full((1, 1), carry[1], jnp.int32), k[:-1, -1:]], axis=0)
    prevk = jnp.concatenate([rowhead, k[:, :-1]], axis=1)
    f = (k != prevk).astype(jnp.float32)
    ci = lax.broadcasted_iota(jnp.int32, (512, 512), 0)
    cj = lax.broadcasted_iota(jnp.int32, (512, 512), 1)
    lower = (ci <= cj).astype(jnp.float32)
    cs = jnp.dot(f, lower, preferred_element_type=jnp.float32)
    rt = cs[:, -1:]
    ri = lax.broadcasted_iota(jnp.int32, (16, 16), 0)
    rj = lax.broadcasted_iota(jnp.int32, (16, 16), 1)
    strict = (rj < ri).astype(jnp.float32)
    ro = jnp.dot(strict, rt, preferred_element_type=jnp.float32)
    incl = cs + ro
    rid_ref[...] = carry[0] + incl.astype(jnp.int32) - 1
    carry[0] = carry[0] + incl[-1, -1].astype(jnp.int32)
    carry[1] = k[-1, -1]


def _run_ids(keys_sorted):
    k2 = keys_sorted.reshape(E2P // 512, 512)
    rid = pl.pallas_call(
        _runid_body,
        grid=(E2P // 8192,),
        in_specs=[pl.BlockSpec((16, 512), lambda i: (i, 0))],
        out_specs=pl.BlockSpec((16, 512), lambda i: (i, 0)),
        out_shape=jax.ShapeDtypeStruct((E2P // 512, 512), jnp.int32),
        scratch_shapes=[pltpu.SMEM((2,), jnp.int32)],
    )(k2)
    return rid.reshape(E2P)


def _run_tables_body(so_hbm, rid_hbm, val_hbm, adj_hbm,
                     vparts_hbm, aparts_hbm, cparts_hbm, ridbyo_hbm,
                     o_v, rid_v, op_v, val_v, ad_v, a2_v, c1_v, zero_v,
                     vsum_sh, asum_sh, cnt_sh):
    cid = lax.axis_index("c")
    sid = lax.axis_index("s")
    wid = sid * 2 + cid

    @pl.loop(0, 2048, step=16)
    def _(j):
        zero_v[pl.ds(j, 16)] = jnp.zeros((16,), jnp.float32)

    zbase = sid * (E2P // 16)

    @pl.loop(0, E2P // 16, step=2048)
    def _(z):
        pltpu.sync_copy(zero_v, vsum_sh.at[pl.ds(zbase + z, 2048)])
        pltpu.sync_copy(zero_v, asum_sh.at[pl.ds(zbase + z, 2048)])
        pltpu.sync_copy(zero_v, cnt_sh.at[pl.ds(zbase + z, 2048)])

    plsc.subcore_barrier()

    base0 = wid * E2PT

    @pl.loop(0, E2PT, step=W)
    def _(off):
        base = base0 + off
        pltpu.sync_copy(so_hbm.at[pl.ds(base, W)], o_v)
        pltpu.sync_copy(rid_hbm.at[pl.ds(base, W)], rid_v)

        @pl.loop(0, W, step=16)
        def _(j):
            ov = o_v[pl.ds(j, 16)]
            opv = jnp.where(ov < NE, ov, ov - NE)
            op_v[pl.ds(j, 16)] = jnp.minimum(opv, NE - 1)

        pltpu.sync_copy(val_hbm.at[op_v], val_v)
        pltpu.sync_copy(adj_hbm.at[op_v], ad_v)

        @pl.loop(0, W, step=16)
        def _(j):
            ov = o_v[pl.ds(j, 16)]
            orig = ov < NE
            a2_v[pl.ds(j, 16)] = jnp.where(orig, ad_v[pl.ds(j, 16)], 0.0)
            c1_v[pl.ds(j, 16)] = jnp.where(orig, 1.0, 0.0)

        pltpu.sync_copy(val_v, vsum_sh.at[rid_v], add=True)
        pltpu.sync_copy(a2_v, asum_sh.at[rid_v], add=True)
        pltpu.sync_copy(c1_v, cnt_sh.at[rid_v], add=True)
        pltpu.sync_copy(rid_v, ridbyo_hbm.at[o_v])

    plsc.subcore_barrier()

    dbase = sid * (E2P // 16)
    obase = cid * E2P + dbase

    @pl.loop(0, E2P // 16, step=2048)
    def _(z):
        pltpu.sync_copy(vsum_sh.at[pl.ds(dbase + z, 2048)],
                        vparts_hbm.at[pl.ds(obase + z, 2048)])
        pltpu.sync_copy(asum_sh.at[pl.ds(dbase + z, 2048)],
                        aparts_hbm.at[pl.ds(obase + z, 2048)])
        pltpu.sync_copy(cnt_sh.at[pl.ds(dbase + z, 2048)],
                        cparts_hbm.at[pl.ds(obase + z, 2048)])


def _run_tables(so, rid, values, adj_data):
    return pl.kernel(
        _run_tables_body,
        out_type=[
            jax.ShapeDtypeStruct((2 * E2P,), jnp.float32),
            jax.ShapeDtypeStruct((2 * E2P,), jnp.float32),
            jax.ShapeDtypeStruct((2 * E2P,), jnp.float32),
            jax.ShapeDtypeStruct((E2P,), jnp.int32),
        ],
        mesh=_sc_mesh(),
        scratch_types=[
            pltpu.VMEM((W,), jnp.int32),
            pltpu.VMEM((W,), jnp.int32),
            pltpu.VMEM((W,), jnp.int32),
            pltpu.VMEM((W,), jnp.float32),
            pltpu.VMEM((W,), jnp.float32),
            pltpu.VMEM((W,), jnp.float32),
            pltpu.VMEM((W,), jnp.float32),
            pltpu.VMEM((2048,), jnp.float32),
            pltpu.VMEM_SHARED((E2P,), jnp.float32),
            pltpu.VMEM_SHARED((E2P,), jnp.float32),
            pltpu.VMEM_SHARED((E2P,), jnp.float32),
        ],
    )(so, rid, values, adj_data)


def _merge_body(v0, v1, a0, a1, c0, c1, vo_ref, ao_ref, co_ref):
    vo_ref[...] = v0[...] + v1[...]
    ao_ref[...] = a0[...] + a1[...]
    co_ref[...] = c0[...] + c1[...]


def _merge_tables(vparts, aparts, cparts):
    r = E2P // 512
    br = 128
    sl = lambda t, i: t[i * E2P:(i + 1) * E2P].reshape(r, 512)
    ins = [sl(vparts, 0), sl(vparts, 1), sl(aparts, 0), sl(aparts, 1),
           sl(cparts, 0), sl(cparts, 1)]
    outs = pl.pallas_call(
        _merge_body,
        grid=(r // br,),
        in_specs=[pl.BlockSpec((br, 512), lambda i: (i, 0))] * 6,
        out_specs=[pl.BlockSpec((br, 512), lambda i: (i, 0))] * 3,
        out_shape=[jax.ShapeDtypeStruct((r, 512), jnp.float32)] * 3,
    )(*ins)
    return tuple(o.reshape(E2P) for o in outs)


def _gather_runsums_body(ridbyo_hbm, vsum_hbm, asum_hbm, cnt_hbm,
                         vse_hbm, ase_hbm, cne_hbm, idx_v, buf_v):
    base0 = _wid() * EPT

    @pl.loop(0, EPT, step=W)
    def _(off):
        base = base0 + off
        pltpu.sync_copy(ridbyo_hbm.at[pl.ds(base, W)], idx_v)
        pltpu.sync_copy(vsum_hbm.at[idx_v], buf_v)
        pltpu.sync_copy(buf_v, vse_hbm.at[pl.ds(base, W)])
        pltpu.sync_copy(asum_hbm.at[idx_v], buf_v)
        pltpu.sync_copy(buf_v, ase_hbm.at[pl.ds(base, W)])
        pltpu.sync_copy(cnt_hbm.at[idx_v], buf_v)
        pltpu.sync_copy(buf_v, cne_hbm.at[pl.ds(base, W)])


def _gather_runsums(ridbyo, vsum, asum, cnt):
    return pl.kernel(
        _gather_runsums_body,
        out_type=[jax.ShapeDtypeStruct((EP,), jnp.float32)] * 3,
        mesh=_sc_mesh(),
        scratch_types=[
            pltpu.VMEM((W,), jnp.int32),
            pltpu.VMEM((W,), jnp.float32),
        ],
    )(ridbyo, vsum, asum, cnt)


def _scale_body(g_ref, row_ref, col_ref, ad_ref, vs_ref, as_ref, cn_ref,
                nid_ref, gcu_ref, tv_ref, tuv_ref):
    vs = vs_ref[...]
    asum = as_ref[...]
    row = row_ref[...]
    col = col_ref[...]
    nd = (row != col).astype(jnp.float32)
    c = ad_ref[...] * vs * 0.5 * nd
    cnt = jnp.maximum(cn_ref[...], 1.0)
    u = jnp.where((asum != 0.0) & (vs != 0.0), nd / cnt, 0.0)
    g = g_ref[:, HD:]
    gcu_ref[:, :HD] = g * c
    gcu_ref[:, HD:] = g * u
    isnid = (col == nid_ref[0, 0]).astype(jnp.float32)
    tv_ref[...] = c * isnid
    tuv_ref[...] = u * isnid


def _scale_messages(ga, rowp, colp, adjp, vse, ase, cne, nid):
    be = 2560
    grid = EP // be
    col1 = lambda a: a.reshape(EP, 1)
    outs = pl.pallas_call(
        _scale_body,
        grid=(grid,),
        in_specs=[
            pl.BlockSpec((be, 2 * HD), lambda i: (i, 0)),
            pl.BlockSpec((be, 1), lambda i: (i, 0)),
            pl.BlockSpec((be, 1), lambda i: (i, 0)),
            pl.BlockSpec((be, 1), lambda i: (i, 0)),
            pl.BlockSpec((be, 1), lambda i: (i, 0)),
            pl.BlockSpec((be, 1), lambda i: (i, 0)),
            pl.BlockSpec((be, 1), lambda i: (i, 0)),
            pl.BlockSpec((1, 1), lambda i: (0, 0)),
        ],
        out_specs=[
            pl.BlockSpec((be, 2 * HD), lambda i: (i, 0)),
            pl.BlockSpec((be, 1), lambda i: (i, 0)),
            pl.BlockSpec((be, 1), lambda i: (i, 0)),
        ],
        out_shape=[
            jax.ShapeDtypeStruct((EP, 2 * HD), jnp.float32),
            jax.ShapeDtypeStruct((EP, 1), jnp.float32),
            jax.ShapeDtypeStruct((EP, 1), jnp.float32),
        ],
    )(ga, col1(rowp.astype(jnp.float32)), col1(colp.astype(jnp.float32)),
      col1(adjp), col1(vse), col1(ase), col1(cne),
      jnp.full((1, 1), nid, jnp.float32))
    return outs


def _scatter_rows_body(gcu_hbm, tv_hbm, tuv_hbm, row_hbm, col_hbm,
                       accu_hbm, t_hbm, tu_hbm,
                       idxc_v, idxr_v, rows_v, sc_v, zrow_v, zflat_v,
                       accu_sh, t_sh, tu_sh):
    cid = lax.axis_index("c")
    sid = lax.axis_index("s")
    wid = sid * 2 + cid

    @pl.loop(0, 16)
    def _(r):
        @pl.loop(0, 2 * HD, step=16)
        def _(j):
            zrow_v[r, pl.ds(j, 16)] = jnp.zeros((16,), jnp.float32)

    @pl.loop(0, 2048, step=16)
    def _(j):
        zflat_v[pl.ds(j, 16)] = jnp.zeros((16,), jnp.float32)

    zr = sid * 624
    zn = jnp.where(sid == 15, 640, 624)

    @pl.loop(0, 640, step=16)
    def _(z):
        @pl.when(z < zn)
        def _():
            pltpu.sync_copy(zrow_v, accu_sh.at[pl.ds(zr + z, 16)])

    @pl.when(sid == 0)
    def _():
        @pl.loop(0, NT, step=2048)
        def _(z):
            pltpu.sync_copy(zflat_v, t_sh.at[pl.ds(z, 2048)])
            pltpu.sync_copy(zflat_v, tu_sh.at[pl.ds(z, 2048)])

    plsc.subcore_barrier()

    base0 = wid * EPT

    @pl.loop(0, EPT, step=W)
    def _(off):
        base = base0 + off
        pltpu.sync_copy(col_hbm.at[pl.ds(base, W)], idxc_v)
        pltpu.sync_copy(row_hbm.at[pl.ds(base, W)], idxr_v)
        pltpu.sync_copy(gcu_hbm.at[pl.ds(base, W)], rows_v)
        pltpu.sync_copy(rows_v, accu_sh.at[idxc_v], add=True)
        pltpu.sync_copy(tv_hbm.at[pl.ds(base, W)], sc_v)
        pltpu.sync_copy(sc_v, t_sh.at[idxr_v], add=True)
        pltpu.sync_copy(tuv_hbm.at[pl.ds(base, W)], sc_v)
        pltpu.sync_copy(sc_v, tu_sh.at[idxr_v], add=True)

    plsc.subcore_barrier()

    @pl.loop(0, 640, step=16)
    def _(z):
        @pl.when(z < zn)
        def _():
            pltpu.sync_copy(accu_sh.at[pl.ds(zr + z, 16)],
                            accu_hbm.at[pl.ds(cid * NN + zr + z, 16)])

    @pl.when(sid == 0)
    def _():
        pltpu.sync_copy(t_sh, t_hbm.at[pl.ds(cid * NT, NT)])
        pltpu.sync_copy(tu_sh, tu_hbm.at[pl.ds(cid * NT, NT)])


def _scatter_rows(gcu, tv, tuv, rowp, colp):
    return pl.kernel(
        _scatter_rows_body,
        out_type=[
            jax.ShapeDtypeStruct((2 * NN, 2 * HD), jnp.float32),
            jax.ShapeDtypeStruct((2 * NT,), jnp.float32),
            jax.ShapeDtypeStruct((2 * NT,), jnp.float32),
        ],
        mesh=_sc_mesh(),
        scratch_types=[
            pltpu.VMEM((W,), jnp.int32),
            pltpu.VMEM((W,), jnp.int32),
            pltpu.VMEM((W, 2 * HD), jnp.float32),
            pltpu.VMEM((W,), jnp.float32),
            pltpu.VMEM((16, 2 * HD), jnp.float32),
            pltpu.VMEM((2048,), jnp.float32),
            pltpu.VMEM_SHARED((NN, 2 * HD), jnp.float32),
            pltpu.VMEM_SHARED((NT,), jnp.float32),
            pltpu.VMEM_SHARED((NT,), jnp.float32),
        ],
    )(gcu, tv.reshape(EP), tuv.reshape(EP), rowp, colp)


def _final_body(accu_ref, t_ref, tu_ref, wg2_ref, out_ref):
    acc = accu_ref[:NN] + accu_ref[NN:]
    agg = acc[:, :HD]
    u1 = acc[:, HD:]
    h1 = jnp.maximum(agg, 0.0)
    h1cf = jnp.maximum(u1 - agg, 0.0)
    t = t_ref[0:1, :NN] + t_ref[1:2, :NN]
    tu = tu_ref[0:1, :NN] + tu_ref[1:2, :NN]
    z1 = jnp.dot(jnp.dot(t, h1, preferred_element_type=jnp.float32),
                 wg2_ref[...], preferred_element_type=jnp.float32)
    z2 = jnp.dot(jnp.dot(tu - t, h1cf, preferred_element_type=jnp.float32),
                 wg2_ref[...], preferred_element_type=jnp.float32)
    z = jnp.concatenate([z1, z2], axis=0)
    z = z - jnp.max(z, axis=1, keepdims=True)
    ez = jnp.exp(z)
    out_ref[...] = ez / jnp.sum(ez, axis=1, keepdims=True)


def _final(accu_parts, t_parts, tu_parts, wg2):
    return pl.pallas_call(
        _final_body,
        in_specs=[
            pl.BlockSpec((2 * NN, 2 * HD), lambda: (0, 0)),
            pl.BlockSpec((2, NT), lambda: (0, 0)),
            pl.BlockSpec((2, NT), lambda: (0, 0)),
            pl.BlockSpec((HD, 7), lambda: (0, 0)),
        ],
        out_specs=pl.BlockSpec((2, 7), lambda: (0, 0)),
        out_shape=jax.ShapeDtypeStruct((2, 7), jnp.float32),
    )(accu_parts, t_parts.reshape(2, NT), tu_parts.reshape(2, NT), wg2)


def kernel(x, adj_row, adj_col, adj_data, embed, W0, b0, W1, b1, Wg1, Wg2,
           nodeid, tmp):
    adj_row = adj_row.astype(jnp.int32)
    adj_col = adj_col.astype(jnp.int32)
    nid = jnp.asarray(nodeid, jnp.int32)

    pad_e = EP - NE
    rowp = jnp.concatenate([adj_row, jnp.zeros((pad_e,), jnp.int32)])
    colp = jnp.concatenate([adj_col, jnp.zeros((pad_e,), jnp.int32)])
    adjp = jnp.concatenate([adj_data, jnp.zeros((pad_e,), jnp.float32)])

    keys2 = jnp.concatenate([
        adj_row * NN + adj_col,
        adj_col * NN + adj_row,
        jnp.full((E2P - NE2,), jnp.iinfo(jnp.int32).max, jnp.int32),
    ])
    o2 = jnp.arange(E2P, dtype=jnp.int32)
    keys_sorted, so = lax.sort((keys2, o2), num_keys=1)

    w0a = W0[:128]
    w0b = W0[128:256]
    w0c = W0[256:]
    px, p2 = _projections(embed, x, w0a, w0b, Wg1)

    ga, g2 = _gather_rows(px, p2, rowp, colp)
    c0row = lax.dynamic_slice(embed, (nid, 0), (1, 128))
    values = _edge_mlp(ga, g2, c0row, w0c, b0.reshape(1, HD),
                       W1, b1.reshape(1, 1)).reshape(EP)

    rid = _run_ids(keys_sorted)
    vparts, aparts, cparts, ridbyo = _run_tables(so, rid, values, adjp)
    vsum, asum, cnt = _merge_tables(vparts, aparts, cparts)

    vse, ase, cne = _gather_runsums(ridbyo[:EP], vsum, asum, cnt)
    gcu, tv, tuv = _scale_messages(ga, rowp, colp, adjp, vse, ase, cne, nid)
    accu_parts, t_parts, tu_parts = _scatter_rows(gcu, tv, tuv, rowp, colp)

    return _final(accu_parts, t_parts, tu_parts, Wg2)


# speedup vs baseline: 6.0365x; 1.4286x over previous
import functools

import jax
import jax.numpy as jnp
from jax import lax
from jax.experimental import pallas as pl
from jax.experimental.pallas import tpu as pltpu
from jax.experimental.pallas import tpu_sc as plsc

NN = 10000
NE = 320000
NE2 = 2 * NE
HD = 64

NT = 10240
NTILES = 32
W = 128
EP = 327680
E2P = 655360
EPT = EP // NTILES
E2PT = E2P // NTILES

@functools.cache
def _sc_mesh():
    return plsc.VectorSubcoreMesh(core_axis_name="c", subcore_axis_name="s",
                                  num_cores=2, num_subcores=16)


def _wid():
    return lax.axis_index("s") * 2 + lax.axis_index("c")


def _proj_body(e_ref, x_ref, w0a_ref, w0b_ref, wg1_ref, px_ref, p2_ref):
    e = e_ref[...]
    px_ref[:, :HD] = jnp.dot(e, w0a_ref[...], preferred_element_type=jnp.float32)
    px_ref[:, HD:] = jnp.dot(x_ref[...], wg1_ref[...],
                             preferred_element_type=jnp.float32)
    p2 = jnp.dot(e, w0b_ref[...], preferred_element_type=jnp.float32)
    p2_ref[:, :HD] = p2
    p2_ref[:, HD:] = p2


def _projections(embed, x, w0a, w0b, wg1):
    bn = 400
    grid = NN // bn
    return pl.pallas_call(
        _proj_body,
        grid=(grid,),
        in_specs=[
            pl.BlockSpec((bn, 128), lambda i: (i, 0)),
            pl.BlockSpec((bn, 128), lambda i: (i, 0)),
            pl.BlockSpec((128, HD), lambda i: (0, 0)),
            pl.BlockSpec((128, HD), lambda i: (0, 0)),
            pl.BlockSpec((128, HD), lambda i: (0, 0)),
        ],
        out_specs=[
            pl.BlockSpec((bn, 2 * HD), lambda i: (i, 0)),
            pl.BlockSpec((bn, 2 * HD), lambda i: (i, 0)),
        ],
        out_shape=[
            jax.ShapeDtypeStruct((NN, 2 * HD), jnp.float32),
            jax.ShapeDtypeStruct((NN, 2 * HD), jnp.float32),
        ],
    )(embed, x, w0a, w0b, wg1)


def _gather_rows_body(px_hbm, p2_hbm, row2_hbm, col2_hbm, ga_hbm, g2_hbm,
                      r0, r1, r2, c0, c1, c2,
                      ga0, ga1, ga2, g20, g21, g22,
                      sil, sgx, sg2, swx, sw2):
    wid = _wid()
    nw = EPT // W
    rbase = wid * nw
    base0 = wid * EPT

    def start_i(wi, rb, cb):
        pltpu.async_copy(row2_hbm.at[rbase + wi], rb, sil)
        pltpu.async_copy(col2_hbm.at[rbase + wi], cb, sil)

    def wait_i(rb, cb):
        pltpu.make_async_copy(row2_hbm.at[rbase], rb, sil).wait()
        pltpu.make_async_copy(col2_hbm.at[rbase], cb, sil).wait()

    def start_g(rb, cb, ga_b, g2_b):
        pltpu.async_copy(px_hbm.at[rb], ga_b, sgx)
        pltpu.async_copy(p2_hbm.at[cb], g2_b, sg2)

    def wait_g(rb, ga_b, g2_b):
        pltpu.make_async_copy(px_hbm.at[rb], ga_b, sgx).wait()
        pltpu.make_async_copy(p2_hbm.at[rb], g2_b, sg2).wait()

    def start_w(wi, ga_b, g2_b):
        dst = pl.ds(base0 + wi * W, W)
        pltpu.async_copy(ga_b, ga_hbm.at[dst], swx)
        pltpu.async_copy(g2_b, g2_hbm.at[dst], sw2)

    def wait_w(ga_b, g2_b):
        pltpu.make_async_copy(ga_b, ga_hbm.at[pl.ds(base0, W)], swx).wait()
        pltpu.make_async_copy(g2_b, g2_hbm.at[pl.ds(base0, W)], sw2).wait()

    start_i(0, r0, c0)
    start_i(1, r1, c1)
    start_i(2, r2, c2)
    wait_i(r0, c0)
    start_g(r0, c0, ga0, g20)
    wait_i(r1, c1)
    start_g(r1, c1, ga1, g21)

    @pl.loop(0, nw - 2, step=3)
    def _(w):
        wait_g(r0, ga0, g20)
        wait_i(r2, c2)
        start_g(r2, c2, ga2, g22)
        start_w(w, ga0, g20)

        @pl.when(w + 3 < nw)
        def _():
            start_i(w + 3, r0, c0)

        wait_g(r1, ga1, g21)
        wait_w(ga0, g20)

        @pl.when(w + 3 < nw)
        def _():
            wait_i(r0, c0)
            start_g(r0, c0, ga0, g20)

        start_w(w + 1, ga1, g21)

        @pl.when(w + 4 < nw)
        def _():
            start_i(w + 4, r1, c1)

        wait_g(r2, ga2, g22)
        wait_w(ga1, g21)

        @pl.when(w + 4 < nw)
        def _():
            wait_i(r1, c1)
            start_g(r1, c1, ga1, g21)

        start_w(w + 2, ga2, g22)

        @pl.when(w + 5 < nw)
        def _():
            start_i(w + 5, r2, c2)

        wait_w(ga2, g22)

    wait_g(r0, ga0, g20)
    start_w(nw - 2, ga0, g20)
    wait_g(r1, ga1, g21)
    start_w(nw - 1, ga1, g21)
    wait_w(ga0, g20)
    wait_w(ga1, g21)


def _gather_rows(px, p2, row2, col2):
    return pl.kernel(
        _gather_rows_body,
        out_type=[
            jax.ShapeDtypeStruct((EP, 2 * HD), jnp.float32),
            jax.ShapeDtypeStruct((EP, 2 * HD), jnp.float32),
        ],
        mesh=_sc_mesh(),
        scratch_types=(
            [pltpu.VMEM((W,), jnp.int32)] * 6
            + [pltpu.VMEM((W, 2 * HD), jnp.float32)] * 6
            + [pltpu.SemaphoreType.DMA] * 5
        ),
    )(px, p2, row2, col2)


def _edge_mlp_body(ga_ref, g2_ref, c0row_ref, w0c_ref, b0_ref, w1_ref, b1_ref,
                   v_ref):
    c0 = jnp.dot(c0row_ref[...], w0c_ref[...],
                 preferred_element_type=jnp.float32) + b0_ref[...]
    h = jnp.maximum(ga_ref[:, :HD] + g2_ref[:, :HD] + c0, 0.0)
    z = jnp.dot(h, w1_ref[...], preferred_element_type=jnp.float32) + b1_ref[...]
    v_ref[...] = jax.nn.sigmoid(z)


def _edge_mlp(ga, g2, c0row, w0c, b0, w1, b1):
    be = 2560
    grid = EP // be
    return pl.pallas_call(
        _edge_mlp_body,
        grid=(grid,),
        in_specs=[
            pl.BlockSpec((be, 2 * HD), lambda i: (i, 0)),
            pl.BlockSpec((be, 2 * HD), lambda i: (i, 0)),
            pl.BlockSpec((1, 128), lambda i: (0, 0)),
            pl.BlockSpec((128, HD), lambda i: (0, 0)),
            pl.BlockSpec((1, HD), lambda i: (0, 0)),
            pl.BlockSpec((HD, 1), lambda i: (0, 0)),
            pl.BlockSpec((1, 1), lambda i: (0, 0)),
        ],
        out_specs=pl.BlockSpec((be, 1), lambda i: (i, 0)),
        out_shape=jax.ShapeDtypeStruct((EP, 1), jnp.float32),
    )(ga, g2, c0row, w0c, b0, w1, b1)


def _runid_body(k_ref, rid_ref, carry):
    step = pl.program_id(0)

    @pl.when(step == 0)
    def _():
        carry[0] = 0
        carry[1] = -1

    k = k_ref[...]
    rowhead = jnp.concatenate(
        [jnp.full((1, 1), carry[1], jnp.int32), k[:-1, -1:]], axis=0)
    prevk = jnp.concatenate([rowhead, k[:, :-1]], axis=1)
    f = (k != prevk).astype(jnp.float32)
    ci = lax.broadcasted_iota(jnp.int32, (512, 512), 0)
    cj = lax.broadcasted_iota(jnp.int32, (512, 512), 1)
    lower = (ci <= cj).astype(jnp.float32)
    cs = jnp.dot(f, lower, preferred_element_type=jnp.float32)
    rt = cs[:, -1:]
    ri = lax.broadcasted_iota(jnp.int32, (16, 16), 0)
    rj = lax.broadcasted_iota(jnp.int32, (16, 16), 1)
    strict = (rj < ri).astype(jnp.float32)
    ro = jnp.dot(strict, rt, preferred_element_type=jnp.float32)
    incl = cs + ro
    rid_ref[...] = carry[0] + incl.astype(jnp.int32) - 1
    carry[0] = carry[0] + incl[-1, -1].astype(jnp.int32)
    carry[1] = k[-1, -1]


def _run_ids(keys_sorted):
    k2 = keys_sorted.reshape(E2P // 512, 512)
    rid = pl.pallas_call(
        _runid_body,
        grid=(E2P // 8192,),
        in_specs=[pl.BlockSpec((16, 512), lambda i: (i, 0))],
        out_specs=pl.BlockSpec((16, 512), lambda i: (i, 0)),
        out_shape=jax.ShapeDtypeStruct((E2P // 512, 512), jnp.int32),
        scratch_shapes=[pltpu.SMEM((2,), jnp.int32)],
    )(k2)
    return rid.reshape(E2P)


def _run_tables_body(so2_hbm, rid2_hbm, val_hbm, adj_hbm,
                     vparts_hbm, aparts_hbm, ridbyo_hbm,
                     oa_v, ob_v, ra_v, rb_v, opa_v, opb_v,
                     va_v, vb_v, aa_v, ab_v, a2a_v, a2b_v, zero_v,
                     sgl, sgv, sga, sso, sad, vsum_sh, asum_sh):
    cid = lax.axis_index("c")
    sid = lax.axis_index("s")
    wid = sid * 2 + cid
    nw = E2PT // W

    @pl.loop(0, 2048, step=16)
    def _(j):
        zero_v[pl.ds(j, 16)] = jnp.zeros((16,), jnp.float32)

    zbase = sid * (E2P // 16)

    @pl.loop(0, E2P // 16, step=2048)
    def _(z):
        pltpu.sync_copy(zero_v, vsum_sh.at[pl.ds(zbase + z, 2048)])
        pltpu.sync_copy(zero_v, asum_sh.at[pl.ds(zbase + z, 2048)])

    rbase = wid * nw

    def start_l(wi, o_b, r_b):
        pltpu.async_copy(so2_hbm.at[rbase + wi], o_b, sgl)
        pltpu.async_copy(rid2_hbm.at[rbase + wi], r_b, sgl)

    def wait_l(o_b, r_b):
        pltpu.make_async_copy(so2_hbm.at[rbase], o_b, sgl).wait()
        pltpu.make_async_copy(rid2_hbm.at[rbase], r_b, sgl).wait()

    def comp_op(o_b, op_b):
        @pl.loop(0, W, step=16)
        def _(j):
            ov = o_b[pl.ds(j, 16)]
            opv = jnp.where(ov < NE, ov, ov - NE)
            op_b[pl.ds(j, 16)] = jnp.minimum(opv, NE - 1)

    def start_g(op_b, v_b, a_b):
        pltpu.async_copy(val_hbm.at[op_b], v_b, sgv)
        pltpu.async_copy(adj_hbm.at[op_b], a_b, sga)

    def wait_g(op_b, v_b, a_b):
        pltpu.make_async_copy(val_hbm.at[op_b], v_b, sgv).wait()
        pltpu.make_async_copy(adj_hbm.at[op_b], a_b, sga).wait()

    def process(o_b, r_b, v_b, a_b, a2_b):
        @pl.loop(0, W, step=16)
        def _(j):
            orig = o_b[pl.ds(j, 16)] < NE
            a2_b[pl.ds(j, 16)] = jnp.where(orig, a_b[pl.ds(j, 16)], 0.0)

        pltpu.async_copy(v_b, vsum_sh.at[r_b], sad, add=True)
        pltpu.async_copy(a2_b, asum_sh.at[r_b], sad, add=True)
        pltpu.async_copy(r_b, ridbyo_hbm.at[o_b], sso)

    def wait_adds(o_b, r_b, v_b, a2_b):
        pltpu.make_async_copy(v_b, vsum_sh.at[r_b], sad).wait()
        pltpu.make_async_copy(a2_b, asum_sh.at[r_b], sad).wait()
        pltpu.make_async_copy(r_b, ridbyo_hbm.at[o_b], sso).wait()

    plsc.subcore_barrier()

    start_l(0, oa_v, ra_v)

    @pl.loop(0, nw, step=2)
    def _(w):
        wait_l(oa_v, ra_v)
        start_l(w + 1, ob_v, rb_v)
        comp_op(oa_v, opa_v)
        start_g(opa_v, va_v, aa_v)
        wait_l(ob_v, rb_v)
        comp_op(ob_v, opb_v)
        start_g(opb_v, vb_v, ab_v)
        wait_g(opa_v, va_v, aa_v)
        process(oa_v, ra_v, va_v, aa_v, a2a_v)
        wait_g(opb_v, vb_v, ab_v)
        process(ob_v, rb_v, vb_v, ab_v, a2b_v)
        wait_adds(oa_v, ra_v, va_v, a2a_v)
        wait_adds(ob_v, rb_v, vb_v, a2b_v)

        @pl.when(w + 2 < nw)
        def _():
            start_l(w + 2, oa_v, ra_v)

    plsc.subcore_barrier()

    dbase = sid * (E2P // 16)
    obase = cid * E2P + dbase

    @pl.loop(0, E2P // 16, step=2048)
    def _(z):
        pltpu.sync_copy(vsum_sh.at[pl.ds(dbase + z, 2048)],
                        vparts_hbm.at[pl.ds(obase + z, 2048)])
        pltpu.sync_copy(asum_sh.at[pl.ds(dbase + z, 2048)],
                        aparts_hbm.at[pl.ds(obase + z, 2048)])


def _run_tables(so2, rid2, values, adj_data):
    return pl.kernel(
        _run_tables_body,
        out_type=[
            jax.ShapeDtypeStruct((2 * E2P,), jnp.float32),
            jax.ShapeDtypeStruct((2 * E2P,), jnp.float32),
            jax.ShapeDtypeStruct((E2P,), jnp.int32),
        ],
        mesh=_sc_mesh(),
        scratch_types=(
            [pltpu.VMEM((W,), jnp.int32)] * 6
            + [pltpu.VMEM((W,), jnp.float32)] * 6
            + [pltpu.VMEM((2048,), jnp.float32),
               pltpu.SemaphoreType.DMA,
               pltpu.SemaphoreType.DMA,
               pltpu.SemaphoreType.DMA,
               pltpu.SemaphoreType.DMA,
               pltpu.SemaphoreType.DMA,
               pltpu.VMEM_SHARED((E2P,), jnp.float32),
               pltpu.VMEM_SHARED((E2P,), jnp.float32)]
        ),
    )(so2, rid2, values, adj_data)


def _merge_body(v0, v1, a0, a1, vo_ref, ao_ref):
    vo_ref[...] = v0[...] + v1[...]
    ao_ref[...] = a0[...] + a1[...]


def _merge_tables(vparts, aparts):
    r = E2P // 512
    br = 128
    sl = lambda t, i: t[i * E2P:(i + 1) * E2P].reshape(r, 512)
    ins = [sl(vparts, 0), sl(vparts, 1), sl(aparts, 0), sl(aparts, 1)]
    outs = pl.pallas_call(
        _merge_body,
        grid=(r // br,),
        in_specs=[pl.BlockSpec((br, 512), lambda i: (i, 0))] * 4,
        out_specs=[pl.BlockSpec((br, 512), lambda i: (i, 0))] * 2,
        out_shape=[jax.ShapeDtypeStruct((r, 512), jnp.float32)] * 2,
    )(*ins)
    return tuple(o.reshape(E2P) for o in outs)


def _gather_runsums_body(rb2_hbm, vsum_hbm, asum_hbm,
                         vse_hbm, ase_hbm,
                         idx_v, v0, v1, a0, a1, sg, sw):
    wid = _wid()
    nw = EPT // W
    pltpu.sync_copy(rb2_hbm.at[pl.ds(wid * nw, nw)], idx_v)
    base0 = wid * EPT

    def start_g(wi, vb, ab):
        pltpu.async_copy(vsum_hbm.at[idx_v.at[wi]], vb, sg)
        pltpu.async_copy(asum_hbm.at[idx_v.at[wi]], ab, sg)

    def wait_g(vb, ab):
        pltpu.make_async_copy(vsum_hbm.at[idx_v.at[0]], vb, sg).wait()
        pltpu.make_async_copy(asum_hbm.at[idx_v.at[0]], ab, sg).wait()

    def start_w(wi, vb, ab):
        d = pl.ds(base0 + wi * W, W)
        pltpu.async_copy(vb, vse_hbm.at[d], sw)
        pltpu.async_copy(ab, ase_hbm.at[d], sw)

    def wait_w(vb, ab):
        d = pl.ds(base0, W)
        pltpu.make_async_copy(vb, vse_hbm.at[d], sw).wait()
        pltpu.make_async_copy(ab, ase_hbm.at[d], sw).wait()

    start_g(0, v0, a0)

    @pl.loop(0, nw, step=2)
    def _(w):
        wait_g(v0, a0)
        start_g(w + 1, v1, a1)
        start_w(w, v0, a0)
        wait_g(v1, a1)
        wait_w(v0, a0)

        @pl.when(w + 2 < nw)
        def _():
            start_g(w + 2, v0, a0)

        start_w(w + 1, v1, a1)
        wait_w(v1, a1)


def _gather_runsums(rb2, vsum, asum):
    return pl.kernel(
        _gather_runsums_body,
        out_type=[jax.ShapeDtypeStruct((EP,), jnp.float32)] * 2,
        mesh=_sc_mesh(),
        scratch_types=[
            pltpu.VMEM((EPT // W, W), jnp.int32),
            pltpu.VMEM((W,), jnp.float32),
            pltpu.VMEM((W,), jnp.float32),
            pltpu.VMEM((W,), jnp.float32),
            pltpu.VMEM((W,), jnp.float32),
            pltpu.SemaphoreType.DMA,
            pltpu.SemaphoreType.DMA,
        ],
    )(rb2, vsum, asum)


def _scale_body(g_ref, row_ref, col_ref, ad_ref, vs_ref, as_ref,
                nid_ref, gcu_ref, tv_ref, tuv_ref):
    vs = vs_ref[...]
    asum = as_ref[...]
    row = row_ref[...]
    col = col_ref[...]
    nd = (row != col).astype(jnp.float32)
    ad = ad_ref[...]
    c = ad * vs * 0.5 * nd
    u = jnp.where((asum != 0.0) & (vs != 0.0),
                  nd * ad / jnp.where(asum != 0.0, asum, 1.0), 0.0)
    g = g_ref[:, HD:]
    gcu_ref[:, :HD] = g * c
    gcu_ref[:, HD:] = g * u
    isnid = (col == nid_ref[0, 0]).astype(jnp.float32)
    tv_ref[...] = c * isnid
    tuv_ref[...] = u * isnid


def _scale_messages(ga, rowp, colp, adjp, vse, ase, nid):
    be = 2560
    grid = EP // be
    col1 = lambda a: a.reshape(EP, 1)
    outs = pl.pallas_call(
        _scale_body,
        grid=(grid,),
        in_specs=[
            pl.BlockSpec((be, 2 * HD), lambda i: (i, 0)),
            pl.BlockSpec((be, 1), lambda i: (i, 0)),
            pl.BlockSpec((be, 1), lambda i: (i, 0)),
            pl.BlockSpec((be, 1), lambda i: (i, 0)),
            pl.BlockSpec((be, 1), lambda i: (i, 0)),
            pl.BlockSpec((be, 1), lambda i: (i, 0)),
            pl.BlockSpec((1, 1), lambda i: (0, 0)),
        ],
        out_specs=[
            pl.BlockSpec((be, 2 * HD), lambda i: (i, 0)),
            pl.BlockSpec((be, 1), lambda i: (i, 0)),
            pl.BlockSpec((be, 1), lambda i: (i, 0)),
        ],
        out_shape=[
            jax.ShapeDtypeStruct((EP, 2 * HD), jnp.float32),
            jax.ShapeDtypeStruct((EP, 1), jnp.float32),
            jax.ShapeDtypeStruct((EP, 1), jnp.float32),
        ],
    )(ga, col1(rowp.astype(jnp.float32)), col1(colp.astype(jnp.float32)),
      col1(adjp), col1(vse), col1(ase),
      jnp.full((1, 1), nid, jnp.float32))
    return outs


def _scatter_rows_body(gcu_hbm, tv2_hbm, tuv2_hbm, row2_hbm, col2_hbm,
                       accu_hbm, t_hbm, tu_hbm,
                       ic0, ic1, ir0, ir1, tb0, tb1, ub0, ub1,
                       r0, r1, zrow_v, zflat_v,
                       sg, accu_sh, t_sh, tu_sh):
    cid = lax.axis_index("c")
    sid = lax.axis_index("s")
    wid = sid * 2 + cid
    nw = EPT // W

    @pl.loop(0, 16)
    def _(r):
        @pl.loop(0, 2 * HD, step=16)
        def _(j):
            zrow_v[r, pl.ds(j, 16)] = jnp.zeros((16,), jnp.float32)

    @pl.loop(0, 2048, step=16)
    def _(j):
        zflat_v[pl.ds(j, 16)] = jnp.zeros((16,), jnp.float32)

    zr = sid * 624
    zn = jnp.where(sid == 15, 640, 624)

    @pl.loop(0, 640, step=16)
    def _(z):
        @pl.when(z < zn)
        def _():
            pltpu.sync_copy(zrow_v, accu_sh.at[pl.ds(zr + z, 16)])

    @pl.when(sid == 0)
    def _():
        @pl.loop(0, NT, step=2048)
        def _(z):
            pltpu.sync_copy(zflat_v, t_sh.at[pl.ds(z, 2048)])
            pltpu.sync_copy(zflat_v, tu_sh.at[pl.ds(z, 2048)])

    rbase = wid * nw
    base0 = wid * EPT

    def start_l(wi, rb, ic, ir, tb, ub):
        pltpu.async_copy(gcu_hbm.at[pl.ds(base0 + wi * W, W)], rb, sg)
        pltpu.async_copy(col2_hbm.at[rbase + wi], ic, sg)
        pltpu.async_copy(row2_hbm.at[rbase + wi], ir, sg)
        pltpu.async_copy(tv2_hbm.at[rbase + wi], tb, sg)
        pltpu.async_copy(tuv2_hbm.at[rbase + wi], ub, sg)

    def wait_l(rb, ic, ir, tb, ub):
        pltpu.make_async_copy(gcu_hbm.at[pl.ds(base0, W)], rb, sg).wait()
        pltpu.make_async_copy(col2_hbm.at[rbase], ic, sg).wait()
        pltpu.make_async_copy(row2_hbm.at[rbase], ir, sg).wait()
        pltpu.make_async_copy(tv2_hbm.at[rbase], tb, sg).wait()
        pltpu.make_async_copy(tuv2_hbm.at[rbase], ub, sg).wait()

    def process(rb, ic, ir, tb, ub):
        pltpu.sync_copy(rb, accu_sh.at[ic], add=True)
        pltpu.sync_copy(tb, t_sh.at[ir], add=True)
        pltpu.sync_copy(ub, tu_sh.at[ir], add=True)

    plsc.subcore_barrier()

    start_l(0, r0, ic0, ir0, tb0, ub0)

    @pl.loop(0, nw, step=2)
    def _(w):
        wait_l(r0, ic0, ir0, tb0, ub0)
        start_l(w + 1, r1, ic1, ir1, tb1, ub1)
        process(r0, ic0, ir0, tb0, ub0)
        wait_l(r1, ic1, ir1, tb1, ub1)

        @pl.when(w + 2 < nw)
        def _():
            start_l(w + 2, r0, ic0, ir0, tb0, ub0)

        process(r1, ic1, ir1, tb1, ub1)

    plsc.subcore_barrier()

    @pl.loop(0, 640, step=16)
    def _(z):
        @pl.when(z < zn)
        def _():
            pltpu.sync_copy(accu_sh.at[pl.ds(zr + z, 16)],
                            accu_hbm.at[pl.ds(cid * NN + zr + z, 16)])

    @pl.when(sid == 0)
    def _():
        pltpu.sync_copy(t_sh, t_hbm.at[pl.ds(cid * NT, NT)])
        pltpu.sync_copy(tu_sh, tu_hbm.at[pl.ds(cid * NT, NT)])


def _scatter_rows(gcu, tv2, tuv2, row2, col2):
    return pl.kernel(
        _scatter_rows_body,
        out_type=[
            jax.ShapeDtypeStruct((2 * NN, 2 * HD), jnp.float32),
            jax.ShapeDtypeStruct((2 * NT,), jnp.float32),
            jax.ShapeDtypeStruct((2 * NT,), jnp.float32),
        ],
        mesh=_sc_mesh(),
        scratch_types=(
            [pltpu.VMEM((W,), jnp.int32)] * 4
            + [pltpu.VMEM((W,), jnp.float32)] * 4
            + [pltpu.VMEM((W, 2 * HD), jnp.float32)] * 2
            + [pltpu.VMEM((16, 2 * HD), jnp.float32),
               pltpu.VMEM((2048,), jnp.float32),
               pltpu.SemaphoreType.DMA,
               pltpu.VMEM_SHARED((NN, 2 * HD), jnp.float32),
               pltpu.VMEM_SHARED((NT,), jnp.float32),
               pltpu.VMEM_SHARED((NT,), jnp.float32)]
        ),
    )(gcu, tv2, tuv2, row2, col2)


def _final_body(accu_ref, t_ref, tu_ref, wg2_ref, out_ref):
    acc = accu_ref[:NN] + accu_ref[NN:]
    agg = acc[:, :HD]
    u1 = acc[:, HD:]
    h1 = jnp.maximum(agg, 0.0)
    h1cf = jnp.maximum(u1 - agg, 0.0)
    t = t_ref[0:1, :NN] + t_ref[1:2, :NN]
    tu = tu_ref[0:1, :NN] + tu_ref[1:2, :NN]
    z1 = jnp.dot(jnp.dot(t, h1, preferred_element_type=jnp.float32),
                 wg2_ref[...], preferred_element_type=jnp.float32)
    z2 = jnp.dot(jnp.dot(tu - t, h1cf, preferred_element_type=jnp.float32),
                 wg2_ref[...], preferred_element_type=jnp.float32)
    z = jnp.concatenate([z1, z2], axis=0)
    z = z - jnp.max(z, axis=1, keepdims=True)
    ez = jnp.exp(z)
    out_ref[...] = ez / jnp.sum(ez, axis=1, keepdims=True)


def _final(accu_parts, t_parts, tu_parts, wg2):
    return pl.pallas_call(
        _final_body,
        in_specs=[
            pl.BlockSpec((2 * NN, 2 * HD), lambda: (0, 0)),
            pl.BlockSpec((2, NT), lambda: (0, 0)),
            pl.BlockSpec((2, NT), lambda: (0, 0)),
            pl.BlockSpec((HD, 7), lambda: (0, 0)),
        ],
        out_specs=pl.BlockSpec((2, 7), lambda: (0, 0)),
        out_shape=jax.ShapeDtypeStruct((2, 7), jnp.float32),
    )(accu_parts, t_parts.reshape(2, NT), tu_parts.reshape(2, NT), wg2)


def kernel(x, adj_row, adj_col, adj_data, embed, W0, b0, W1, b1, Wg1, Wg2,
           nodeid, tmp):
    adj_row = adj_row.astype(jnp.int32)
    adj_col = adj_col.astype(jnp.int32)
    nid = jnp.asarray(nodeid, jnp.int32)

    pad_e = EP - NE
    rowp = jnp.concatenate([adj_row, jnp.zeros((pad_e,), jnp.int32)])
    colp = jnp.concatenate([adj_col, jnp.zeros((pad_e,), jnp.int32)])
    adjp = jnp.concatenate([adj_data, jnp.zeros((pad_e,), jnp.float32)])

    keys2 = jnp.concatenate([
        adj_row * NN + adj_col,
        adj_col * NN + adj_row,
        jnp.full((E2P - NE2,), jnp.iinfo(jnp.int32).max, jnp.int32),
    ])
    o2 = jnp.arange(E2P, dtype=jnp.int32)
    keys_sorted, so = lax.sort((keys2, o2), num_keys=1)

    w0a = W0[:128]
    w0b = W0[128:256]
    w0c = W0[256:]
    px, p2 = _projections(embed, x, w0a, w0b, Wg1)

    row2 = rowp.reshape(EP // W, W)
    col2 = colp.reshape(EP // W, W)
    ga, g2 = _gather_rows(px, p2, row2, col2)
    c0row = lax.dynamic_slice(embed, (nid, 0), (1, 128))
    values = _edge_mlp(ga, g2, c0row, w0c, b0.reshape(1, HD),
                       W1, b1.reshape(1, 1)).reshape(EP)

    rid = _run_ids(keys_sorted)
    vparts, aparts, ridbyo = _run_tables(
        so.reshape(E2P // W, W), rid.reshape(E2P // W, W), values, adjp)
    vsum, asum = _merge_tables(vparts, aparts)

    vse, ase = _gather_runsums(ridbyo.reshape(E2P // W, W), vsum, asum)
    gcu, tv, tuv = _scale_messages(ga, rowp, colp, adjp, vse, ase, nid)
    accu_parts, t_parts, tu_parts = _scatter_rows(
        gcu, tv.reshape(EP // W, W), tuv.reshape(EP // W, W), row2, col2)

    return _final(accu_parts, t_parts, tu_parts, Wg2)
